# baseline (speedup 1.0000x reference)
import functools
import math

import jax
import jax.numpy as jnp
from jax import lax
from jax.experimental import pallas as pl
from jax.experimental.pallas import tpu as pltpu
from jax.experimental.pallas import tpu_sc as plsc

N = 10000
E = 320000
H = 128
NHEADS = 4
HD = H // NHEADS

NC = 2
NS = 16
NW = NC * NS
CHUNK = 128
N_CHUNKS = E // CHUNK
CH_PER_W = -(-N_CHUNKS // NW)
NP = 10240
ROWS_PER_TILE = NP // NS


def _mesh():
    return plsc.VectorSubcoreMesh(core_axis_name="c", subcore_axis_name="s")


NBUF = 2
TOT_STEPS = -(-CH_PER_W // NBUF) * NBUF


def _gather_sc(x, row, col):
    @functools.partial(
        pl.kernel,
        mesh=_mesh(),
        out_type=(jax.ShapeDtypeStruct((E, H), jnp.float32),
                  jax.ShapeDtypeStruct((E, H), jnp.float32)),
        scratch_types=[
            pltpu.VMEM((CHUNK,), jnp.int32), pltpu.VMEM((CHUNK,), jnp.int32),
            pltpu.VMEM((CHUNK,), jnp.int32), pltpu.VMEM((CHUNK,), jnp.int32),
            pltpu.VMEM((CHUNK, H), jnp.float32), pltpu.VMEM((CHUNK, H), jnp.float32),
            pltpu.VMEM((CHUNK, H), jnp.float32), pltpu.VMEM((CHUNK, H), jnp.float32),
        ] + [pltpu.SemaphoreType.DMA] * 12,
    )
    def k(x_hbm, row_hbm, col_hbm, xr_hbm, xc_hbm,
          ir0, ir1, ic0, ic1, rr0, rr1, rc0, rc1,
          sir0, sir1, sic0, sic1, sgr0, sgr1, sgc0, sgc1,
          swr0, swr1, swc0, swc1):
        idx_r = [ir0, ir1]
        idx_c = [ic0, ic1]
        rows_r = [rr0, rr1]
        rows_c = [rc0, rc1]
        sem_ir = [sir0, sir1]
        sem_ic = [sic0, sic1]
        sem_gr = [sgr0, sgr1]
        sem_gc = [sgc0, sgc1]
        sem_wr = [swr0, swr1]
        sem_wc = [swc0, swc1]
        cid = lax.axis_index("c")
        sid = lax.axis_index("s")
        wid = sid * NC + cid

        def chunk_base(j):
            return pl.multiple_of((wid + j * NW) * CHUNK, 8)

        b0 = chunk_base(0)
        pltpu.async_copy(row_hbm.at[pl.ds(b0, CHUNK)], idx_r[0], sem_ir[0])
        pltpu.async_copy(col_hbm.at[pl.ds(b0, CHUNK)], idx_c[0], sem_ic[0])

        def step(j, b):
            ci = wid + j * NW
            valid = ci < N_CHUNKS
            nb = (b + 1) % NBUF

            @pl.when(valid)
            def _():
                pltpu.make_async_copy(row_hbm.at[pl.ds(chunk_base(j), CHUNK)],
                                      idx_r[b], sem_ir[b]).wait()
                pltpu.make_async_copy(col_hbm.at[pl.ds(chunk_base(j), CHUNK)],
                                      idx_c[b], sem_ic[b]).wait()

            @pl.when(ci + NW < N_CHUNKS)
            def _():
                nbase = chunk_base(j + 1)
                pltpu.async_copy(row_hbm.at[pl.ds(nbase, CHUNK)], idx_r[nb], sem_ir[nb])
                pltpu.async_copy(col_hbm.at[pl.ds(nbase, CHUNK)], idx_c[nb], sem_ic[nb])

            pv = ci - NBUF * NW

            @pl.when((pv >= 0) & (pv < N_CHUNKS))
            def _():
                pbase = chunk_base(j - NBUF)
                pltpu.make_async_copy(rows_r[b], xr_hbm.at[pl.ds(pbase, CHUNK)],
                                      sem_wr[b]).wait()
                pltpu.make_async_copy(rows_c[b], xc_hbm.at[pl.ds(pbase, CHUNK)],
                                      sem_wc[b]).wait()

            @pl.when(valid)
            def _():
                base = chunk_base(j)
                pltpu.async_copy(x_hbm.at[idx_r[b]], rows_r[b], sem_gr[b])
                pltpu.async_copy(x_hbm.at[idx_c[b]], rows_c[b], sem_gc[b])
                pltpu.make_async_copy(x_hbm.at[idx_r[b]], rows_r[b], sem_gr[b]).wait()
                pltpu.make_async_copy(x_hbm.at[idx_c[b]], rows_c[b], sem_gc[b]).wait()
                pltpu.async_copy(rows_r[b], xr_hbm.at[pl.ds(base, CHUNK)], sem_wr[b])
                pltpu.async_copy(rows_c[b], xc_hbm.at[pl.ds(base, CHUNK)], sem_wc[b])

        def body(jj, carry):
            for b in range(NBUF):
                step(jj * NBUF + b, b)
            return carry

        lax.fori_loop(0, TOT_STEPS // NBUF, body, 0)
        for b in range(NBUF):
            j = TOT_STEPS - NBUF + b
            ci = wid + j * NW

            @pl.when(ci < N_CHUNKS)
            def _():
                base = chunk_base(j)
                pltpu.make_async_copy(rows_r[b], xr_hbm.at[pl.ds(base, CHUNK)],
                                      sem_wr[b]).wait()
                pltpu.make_async_copy(rows_c[b], xc_hbm.at[pl.ds(base, CHUNK)],
                                      sem_wc[b]).wait()

    return k(x, row, col)


def _scatter_sc(vals, col, zeros_nh):
    @functools.partial(
        pl.kernel,
        mesh=_mesh(),
        out_type=jax.ShapeDtypeStruct((NC, NP, H), jnp.float32),
        scratch_types=[
            pltpu.VMEM_SHARED((NP, H), jnp.float32),
            pltpu.VMEM((CHUNK,), jnp.int32), pltpu.VMEM((CHUNK,), jnp.int32),
            pltpu.VMEM((CHUNK, H), jnp.float32), pltpu.VMEM((CHUNK, H), jnp.float32),
        ] + [pltpu.SemaphoreType.DMA] * 6,
    )
    def k(vals_hbm, col_hbm, zeros_hbm, out_hbm, acc, idx0, idx1, val0, val1,
          si0, si1, sv0, sv1, ss0, ss1):
        idx_b = [idx0, idx1]
        val_b = [val0, val1]
        sem_i = [si0, si1]
        sem_v = [sv0, sv1]
        sem_s = [ss0, ss1]
        cid = lax.axis_index("c")
        sid = lax.axis_index("s")
        wid = sid * NC + cid
        rbase = pl.multiple_of(sid * ROWS_PER_TILE, 8)
        pltpu.sync_copy(zeros_hbm.at[pl.ds(rbase, ROWS_PER_TILE)],
                        acc.at[pl.ds(rbase, ROWS_PER_TILE)])
        plsc.subcore_barrier()

        def chunk_base(j):
            return pl.multiple_of((wid + j * NW) * CHUNK, 8)

        b0 = chunk_base(0)
        pltpu.async_copy(col_hbm.at[pl.ds(b0, CHUNK)], idx_b[0], sem_i[0])
        pltpu.async_copy(vals_hbm.at[pl.ds(b0, CHUNK)], val_b[0], sem_v[0])

        def step(j, b):
            ci = wid + j * NW
            valid = ci < N_CHUNKS
            nb = (b + 1) % NBUF

            @pl.when(valid)
            def _():
                base = chunk_base(j)
                pltpu.make_async_copy(col_hbm.at[pl.ds(base, CHUNK)],
                                      idx_b[b], sem_i[b]).wait()
                pltpu.make_async_copy(vals_hbm.at[pl.ds(base, CHUNK)],
                                      val_b[b], sem_v[b]).wait()
                pltpu.async_copy(val_b[b], acc.at[idx_b[b]], sem_s[b], add=True)

            pv = ci - NW

            @pl.when((pv >= 0) & (pv < N_CHUNKS))
            def _():
                pltpu.make_async_copy(val_b[nb], acc.at[idx_b[nb]],
                                      sem_s[nb]).wait()

            @pl.when(ci + NW < N_CHUNKS)
            def _():
                nbase = chunk_base(j + 1)
                pltpu.async_copy(col_hbm.at[pl.ds(nbase, CHUNK)], idx_b[nb], sem_i[nb])
                pltpu.async_copy(vals_hbm.at[pl.ds(nbase, CHUNK)], val_b[nb], sem_v[nb])

        def body(jj, carry):
            for b in range(NBUF):
                step(jj * NBUF + b, b)
            return carry

        lax.fori_loop(0, TOT_STEPS // NBUF, body, 0)
        j_last = TOT_STEPS - 1
        ci_last = wid + j_last * NW
        b_last = j_last % NBUF

        @pl.when(ci_last < N_CHUNKS)
        def _():
            pltpu.make_async_copy(val_b[b_last], acc.at[idx_b[b_last]],
                                  sem_s[b_last]).wait()

        plsc.subcore_barrier()
        pltpu.sync_copy(acc.at[pl.ds(rbase, ROWS_PER_TILE)],
                        out_hbm.at[cid, pl.ds(rbase, ROWS_PER_TILE)])

    return k(vals, col, zeros_nh)


def _count_sc(col, ones_ch, zeros_nh):
    @functools.partial(
        pl.kernel,
        mesh=_mesh(),
        out_type=jax.ShapeDtypeStruct((NC, NP, H), jnp.float32),
        scratch_types=[
            pltpu.VMEM_SHARED((NP, H), jnp.float32),
            pltpu.VMEM((CHUNK,), jnp.int32), pltpu.VMEM((CHUNK,), jnp.int32),
            pltpu.VMEM((CHUNK, H), jnp.float32),
        ] + [pltpu.SemaphoreType.DMA] * 4,
    )
    def k(col_hbm, ones_hbm, zeros_hbm, out_hbm, acc, idx0, idx1, ones_v,
          si0, si1, ss0, ss1):
        idx_b = [idx0, idx1]
        sem_i = [si0, si1]
        sem_s = [ss0, ss1]
        cid = lax.axis_index("c")
        sid = lax.axis_index("s")
        wid = sid * NC + cid
        rbase = pl.multiple_of(sid * ROWS_PER_TILE, 8)
        pltpu.sync_copy(zeros_hbm.at[pl.ds(rbase, ROWS_PER_TILE)],
                        acc.at[pl.ds(rbase, ROWS_PER_TILE)])
        pltpu.sync_copy(ones_hbm, ones_v)
        plsc.subcore_barrier()

        def chunk_base(j):
            return pl.multiple_of((wid + j * NW) * CHUNK, 8)

        pltpu.async_copy(col_hbm.at[pl.ds(chunk_base(0), CHUNK)], idx_b[0], sem_i[0])

        def step(j, b):
            ci = wid + j * NW
            nb = (b + 1) % NBUF

            @pl.when(ci < N_CHUNKS)
            def _():
                pltpu.make_async_copy(col_hbm.at[pl.ds(chunk_base(j), CHUNK)],
                                      idx_b[b], sem_i[b]).wait()
                pltpu.async_copy(ones_v, acc.at[idx_b[b]], sem_s[b], add=True)

            pv = ci - NW

            @pl.when((pv >= 0) & (pv < N_CHUNKS))
            def _():
                pltpu.make_async_copy(ones_v, acc.at[idx_b[nb]], sem_s[nb]).wait()

            @pl.when(ci + NW < N_CHUNKS)
            def _():
                pltpu.async_copy(col_hbm.at[pl.ds(chunk_base(j + 1), CHUNK)],
                                 idx_b[nb], sem_i[nb])

        def body(jj, carry):
            for b in range(NBUF):
                step(jj * NBUF + b, b)
            return carry

        lax.fori_loop(0, TOT_STEPS // NBUF, body, 0)
        j_last = TOT_STEPS - 1
        ci_last = wid + j_last * NW
        b_last = j_last % NBUF

        @pl.when(ci_last < N_CHUNKS)
        def _():
            pltpu.make_async_copy(ones_v, acc.at[idx_b[b_last]], sem_s[b_last]).wait()

        plsc.subcore_barrier()
        pltpu.sync_copy(acc.at[pl.ds(rbase, ROWS_PER_TILE)],
                        out_hbm.at[cid, pl.ds(rbase, ROWS_PER_TILE)])

    return k(col, ones_ch, zeros_nh)


def _ln(h, g, b):
    m = jnp.mean(h, axis=-1, keepdims=True)
    v = jnp.mean(jnp.square(h - m), axis=-1, keepdims=True)
    return (h - m) * lax.rsqrt(v + 1e-5) * g + b


def _gelu(h):
    return 0.5 * h * (1.0 + lax.erf(h * (1.0 / math.sqrt(2.0))))


E_B = 2000


def _edge_tc(xr, xc, ea, p):
    W1 = p["lin1"]["W"]
    w1a, w1b, w1c = W1[:H], W1[H:2 * H], W1[2 * H:]
    b1 = p["lin1"]["b"][None, :]
    w2 = p["lin2"]["W"]
    b2 = p["lin2"]["b"][None, :]
    g1, be1 = p["ln1_g"][None, :], p["ln1_b"][None, :]
    g2, be2 = p["ln2_g"][None, :], p["ln2_b"][None, :]

    def body(xr_r, xc_r, ea_r, w1a_r, w1b_r, w1c_r, b1_r, g1_r, be1_r,
             w2_r, b2_r, g2_r, be2_r, out_r):
        bf = jnp.bfloat16
        h = jnp.dot(xr_r[...].astype(bf), w1a_r[...].astype(bf),
                    preferred_element_type=jnp.float32)
        h = h + jnp.dot(xc_r[...].astype(bf), w1b_r[...].astype(bf),
                        preferred_element_type=jnp.float32)
        h = h + jnp.dot(ea_r[...].astype(bf), w1c_r[...].astype(bf),
                        preferred_element_type=jnp.float32)
        h = h + b1_r[...]
        h = _gelu(_ln(h, g1_r[...], be1_r[...]))
        h2 = jnp.dot(h.astype(bf), w2_r[...].astype(bf),
                     preferred_element_type=jnp.float32) + b2_r[...]
        h2 = _ln(h2, g2_r[...], be2_r[...])
        out_r[...] = ea_r[...] + h2

    eb = pl.BlockSpec((E_B, H), lambda i: (i, 0))
    hb = pl.BlockSpec((E_B, 2 * H), lambda i: (i, 0))
    full = lambda a: pl.BlockSpec(a.shape, lambda i: tuple(0 for _ in a.shape))
    return pl.pallas_call(
        body,
        grid=(E // E_B,),
        in_specs=[eb, eb, eb, full(w1a), full(w1b), full(w1c), full(b1),
                  full(g1), full(be1), full(w2), full(b2), full(g2), full(be2)],
        out_specs=eb,
        out_shape=jax.ShapeDtypeStruct((E, H), jnp.float32),
    )(xr, xc, ea, w1a, w1b, w1c, b1, g1, be1, w2, b2, g2, be2)


def _tokens_tc(x, p):
    te = p["token_embed"]
    a = p["attn_tok"]
    args = [x, te,
            a["q"]["W"], a["q"]["b"][None, :], a["k"]["W"], a["k"]["b"][None, :],
            a["v"]["W"], a["v"]["b"][None, :], a["o"]["W"], a["o"]["b"][None, :],
            p["ln_tok1_g"][None, :], p["ln_tok1_b"][None, :],
            p["ff1"]["W"], p["ff1"]["b"][None, :],
            p["ff2"]["W"], p["ff2"]["b"][None, :],
            p["ln_tok2_g"][None, :], p["ln_tok2_b"][None, :]]

    def body(x_r, te_r, wq, bq, wk, bk, wv, bv, wo, bo, g1, be1,
             wf1, bf1, wf2, bf2, g2, be2, out_r):
        xx = x_r[...]
        tok = te_r[...]
        q = jnp.dot(tok, wq[...], preferred_element_type=jnp.float32) + bq[...]
        kk = jnp.dot(xx, wk[...], preferred_element_type=jnp.float32) + bk[...]
        vv = jnp.dot(xx, wv[...], preferred_element_type=jnp.float32) + bv[...]
        outs = []
        scale = 1.0 / math.sqrt(float(HD))
        for hh in range(NHEADS):
            sl = slice(hh * HD, (hh + 1) * HD)
            logits = lax.dot_general(q[:, sl], kk[:, sl],
                                     (((1,), (1,)), ((), ()))) * scale
            m = jnp.max(logits, axis=-1, keepdims=True)
            ex = jnp.exp(logits - m)
            pr = ex / jnp.sum(ex, axis=-1, keepdims=True)
            outs.append(lax.dot_general(pr, vv[:, sl], (((1,), (0,)), ((), ()))))
        o = jnp.concatenate(outs, axis=1)
        tok = tok + jnp.dot(o, wo[...], preferred_element_type=jnp.float32) + bo[...]
        tok = _ln(tok, g1[...], be1[...])
        f = _gelu(jnp.dot(tok, wf1[...], preferred_element_type=jnp.float32) + bf1[...])
        tok = tok + jnp.dot(f, wf2[...], preferred_element_type=jnp.float32) + bf2[...]
        tok = _ln(tok, g2[...], be2[...])
        out_r[...] = tok

    full = lambda arr: pl.BlockSpec(arr.shape, lambda: tuple(0 for _ in arr.shape))
    return pl.pallas_call(
        body,
        in_specs=[full(a_) for a_ in args],
        out_shape=jax.ShapeDtypeStruct((2, H), jnp.float32),
    )(*args)


N_B = 2000


def _node_tc(x, parts, cnt_parts, tokens, p_attn, p_mlp):
    W1 = p_mlp["lin1"]["W"]
    w1a, w1b, w1c = W1[:H], W1[H:2 * H], W1[2 * H:]
    args = [x, parts[0], parts[1], cnt_parts[0], cnt_parts[1], tokens,
            p_attn["q"]["W"], p_attn["q"]["b"][None, :],
            p_attn["k"]["W"], p_attn["k"]["b"][None, :],
            p_attn["v"]["W"], p_attn["v"]["b"][None, :],
            p_attn["o"]["W"], p_attn["o"]["b"][None, :],
            w1a, w1b, w1c, p_mlp["lin1"]["b"][None, :],
            p_mlp["ln1_g"][None, :], p_mlp["ln1_b"][None, :],
            p_mlp["lin2"]["W"], p_mlp["lin2"]["b"][None, :],
            p_mlp["ln2_g"][None, :], p_mlp["ln2_b"][None, :]]

    def body(x_r, p0_r, p1_r, c0_r, c1_r, tok_r,
             wq, bq, wk, bk, wv, bv, wo, bo,
             w1a_r, w1b_r, w1c_r, b1_r, g1, be1, w2, b2, g2, be2, out_r):
        xx = x_r[...]
        cnt = c0_r[...][:, 0:1] + c1_r[...][:, 0:1]
        agg = (p0_r[...] + p1_r[...]) / jnp.maximum(cnt, 1.0)
        tok = tok_r[...]
        q = jnp.dot(xx, wq[...], preferred_element_type=jnp.float32) + bq[...]
        tk = jnp.dot(tok, wk[...], preferred_element_type=jnp.float32) + bk[...]
        tv = jnp.dot(tok, wv[...], preferred_element_type=jnp.float32) + bv[...]
        scale = 1.0 / math.sqrt(float(HD))
        outs = []
        for hh in range(NHEADS):
            sl = slice(hh * HD, (hh + 1) * HD)
            logits = lax.dot_general(q[:, sl], tk[:, sl],
                                     (((1,), (1,)), ((), ()))) * scale
            m = jnp.max(logits, axis=-1, keepdims=True)
            ex = jnp.exp(logits - m)
            pr = ex / jnp.sum(ex, axis=-1, keepdims=True)
            outs.append(lax.dot_general(pr, tv[:, sl], (((1,), (0,)), ((), ()))))
        ctx = jnp.dot(jnp.concatenate(outs, axis=1), wo[...],
                      preferred_element_type=jnp.float32) + bo[...]
        h = jnp.dot(xx, w1a_r[...], preferred_element_type=jnp.float32)
        h = h + jnp.dot(agg, w1b_r[...], preferred_element_type=jnp.float32)
        h = h + jnp.dot(ctx, w1c_r[...], preferred_element_type=jnp.float32)
        h = h + b1_r[...]
        h = _gelu(_ln(h, g1[...], be1[...]))
        h2 = jnp.dot(h, w2[...], preferred_element_type=jnp.float32) + b2[...]
        h2 = _ln(h2, g2[...], be2[...])
        out_r[...] = xx + h2

    nb = pl.BlockSpec((N_B, H), lambda i: (i, 0))
    cb = pl.BlockSpec((N_B, H), lambda i: (i, 0))
    full = lambda arr: pl.BlockSpec(arr.shape, lambda i: tuple(0 for _ in arr.shape))
    specs = [nb, nb, nb, cb, cb] + [full(a_) for a_ in args[5:]]
    return pl.pallas_call(
        body,
        grid=(N // N_B,),
        in_specs=specs,
        out_specs=nb,
        out_shape=jax.ShapeDtypeStruct((N, H), jnp.float32),
    )(*args)


def kernel(x, edge_index, edge_attr, params):
    row = edge_index[0]
    col = edge_index[1]
    zeros_nh = jnp.zeros((NP, H), jnp.float32)
    ones_ch = jnp.ones((CHUNK, H), jnp.float32)

    cnt_parts = _count_sc(col, ones_ch, zeros_nh)
    for i in range(2):
        xp = jnp.pad(x, ((0, NP - N), (0, 0)))
        xr, xc = _gather_sc(xp, row, col)
        edge_attr = _edge_tc(xr, xc, edge_attr, params["edge"][i])
        parts = _scatter_sc(edge_attr, col, zeros_nh)
        tokens = _tokens_tc(x, params["gtt"])
        x = _node_tc(x, parts, cnt_parts, tokens,
                     params["gtt"]["attn_node"], params["node"][i])
    return x, edge_attr


# speedup vs baseline: 1.0174x; 1.0174x over previous
import functools
import math

import jax
import jax.numpy as jnp
from jax import lax
from jax.experimental import pallas as pl
from jax.experimental.pallas import tpu as pltpu
from jax.experimental.pallas import tpu_sc as plsc

N = 10000
E = 320000
H = 128
NHEADS = 4
HD = H // NHEADS

NC = 2
NS = 16
NW = NC * NS
CHUNK = 128
N_CHUNKS = E // CHUNK
CH_PER_W = -(-N_CHUNKS // NW)
NP = 10240
ROWS_PER_TILE = NP // NS


def _mesh():
    return plsc.VectorSubcoreMesh(core_axis_name="c", subcore_axis_name="s")


NBUF = 2
TOT_STEPS = -(-CH_PER_W // NBUF) * NBUF


def _gather_sc(x, row, col):
    @functools.partial(
        pl.kernel,
        mesh=_mesh(),
        out_type=(jax.ShapeDtypeStruct((E, H), jnp.float32),
                  jax.ShapeDtypeStruct((E, H), jnp.float32)),
        scratch_types=[
            pltpu.VMEM((CHUNK,), jnp.int32), pltpu.VMEM((CHUNK,), jnp.int32),
            pltpu.VMEM((CHUNK,), jnp.int32), pltpu.VMEM((CHUNK,), jnp.int32),
            pltpu.VMEM((CHUNK, H), jnp.float32), pltpu.VMEM((CHUNK, H), jnp.float32),
            pltpu.VMEM((CHUNK, H), jnp.float32), pltpu.VMEM((CHUNK, H), jnp.float32),
        ] + [pltpu.SemaphoreType.DMA] * 12,
    )
    def k(x_hbm, row_hbm, col_hbm, xr_hbm, xc_hbm,
          ir0, ir1, ic0, ic1, rr0, rr1, rc0, rc1,
          sir0, sir1, sic0, sic1, sgr0, sgr1, sgc0, sgc1,
          swr0, swr1, swc0, swc1):
        idx_r = [ir0, ir1]
        idx_c = [ic0, ic1]
        rows_r = [rr0, rr1]
        rows_c = [rc0, rc1]
        sem_ir = [sir0, sir1]
        sem_ic = [sic0, sic1]
        sem_gr = [sgr0, sgr1]
        sem_gc = [sgc0, sgc1]
        sem_wr = [swr0, swr1]
        sem_wc = [swc0, swc1]
        cid = lax.axis_index("c")
        sid = lax.axis_index("s")
        wid = sid * NC + cid

        def chunk_base(j):
            return pl.multiple_of((wid + j * NW) * CHUNK, 8)

        b0 = chunk_base(0)
        pltpu.async_copy(row_hbm.at[pl.ds(b0, CHUNK)], idx_r[0], sem_ir[0])
        pltpu.async_copy(col_hbm.at[pl.ds(b0, CHUNK)], idx_c[0], sem_ic[0])

        def step(j, b):
            ci = wid + j * NW
            valid = ci < N_CHUNKS
            nb = (b + 1) % NBUF

            @pl.when(valid)
            def _():
                pltpu.make_async_copy(row_hbm.at[pl.ds(chunk_base(j), CHUNK)],
                                      idx_r[b], sem_ir[b]).wait()
                pltpu.make_async_copy(col_hbm.at[pl.ds(chunk_base(j), CHUNK)],
                                      idx_c[b], sem_ic[b]).wait()

            @pl.when(ci + NW < N_CHUNKS)
            def _():
                nbase = chunk_base(j + 1)
                pltpu.async_copy(row_hbm.at[pl.ds(nbase, CHUNK)], idx_r[nb], sem_ir[nb])
                pltpu.async_copy(col_hbm.at[pl.ds(nbase, CHUNK)], idx_c[nb], sem_ic[nb])

            pv = ci - NBUF * NW

            @pl.when((pv >= 0) & (pv < N_CHUNKS))
            def _():
                pbase = chunk_base(j - NBUF)
                pltpu.make_async_copy(rows_r[b], xr_hbm.at[pl.ds(pbase, CHUNK)],
                                      sem_wr[b]).wait()
                pltpu.make_async_copy(rows_c[b], xc_hbm.at[pl.ds(pbase, CHUNK)],
                                      sem_wc[b]).wait()

            @pl.when(valid)
            def _():
                base = chunk_base(j)
                pltpu.async_copy(x_hbm.at[idx_r[b]], rows_r[b], sem_gr[b])
                pltpu.async_copy(x_hbm.at[idx_c[b]], rows_c[b], sem_gc[b])
                pltpu.make_async_copy(x_hbm.at[idx_r[b]], rows_r[b], sem_gr[b]).wait()
                pltpu.make_async_copy(x_hbm.at[idx_c[b]], rows_c[b], sem_gc[b]).wait()
                pltpu.async_copy(rows_r[b], xr_hbm.at[pl.ds(base, CHUNK)], sem_wr[b])
                pltpu.async_copy(rows_c[b], xc_hbm.at[pl.ds(base, CHUNK)], sem_wc[b])

        def body(jj, carry):
            for b in range(NBUF):
                step(jj * NBUF + b, b)
            return carry

        lax.fori_loop(0, TOT_STEPS // NBUF, body, 0)
        for b in range(NBUF):
            j = TOT_STEPS - NBUF + b
            ci = wid + j * NW

            @pl.when(ci < N_CHUNKS)
            def _():
                base = chunk_base(j)
                pltpu.make_async_copy(rows_r[b], xr_hbm.at[pl.ds(base, CHUNK)],
                                      sem_wr[b]).wait()
                pltpu.make_async_copy(rows_c[b], xc_hbm.at[pl.ds(base, CHUNK)],
                                      sem_wc[b]).wait()

    return k(x, row, col)


def _scatter_sc(vals, col, zeros_nh):
    @functools.partial(
        pl.kernel,
        mesh=_mesh(),
        out_type=jax.ShapeDtypeStruct((NC, NP, H), jnp.float32),
        scratch_types=[
            pltpu.VMEM_SHARED((NP, H), jnp.float32),
            pltpu.VMEM((CHUNK,), jnp.int32), pltpu.VMEM((CHUNK,), jnp.int32),
            pltpu.VMEM((CHUNK, H), jnp.float32), pltpu.VMEM((CHUNK, H), jnp.float32),
        ] + [pltpu.SemaphoreType.DMA] * 6,
    )
    def k(vals_hbm, col_hbm, zeros_hbm, out_hbm, acc, idx0, idx1, val0, val1,
          si0, si1, sv0, sv1, ss0, ss1):
        idx_b = [idx0, idx1]
        val_b = [val0, val1]
        sem_i = [si0, si1]
        sem_v = [sv0, sv1]
        sem_s = [ss0, ss1]
        cid = lax.axis_index("c")
        sid = lax.axis_index("s")
        wid = sid * NC + cid
        rbase = pl.multiple_of(sid * ROWS_PER_TILE, 8)
        pltpu.sync_copy(zeros_hbm.at[pl.ds(rbase, ROWS_PER_TILE)],
                        acc.at[pl.ds(rbase, ROWS_PER_TILE)])
        plsc.subcore_barrier()

        def chunk_base(j):
            return pl.multiple_of((wid + j * NW) * CHUNK, 8)

        b0 = chunk_base(0)
        pltpu.async_copy(col_hbm.at[pl.ds(b0, CHUNK)], idx_b[0], sem_i[0])
        pltpu.async_copy(vals_hbm.at[pl.ds(b0, CHUNK)], val_b[0], sem_v[0])

        def step(j, b):
            ci = wid + j * NW
            valid = ci < N_CHUNKS
            nb = (b + 1) % NBUF

            @pl.when(valid)
            def _():
                base = chunk_base(j)
                pltpu.make_async_copy(col_hbm.at[pl.ds(base, CHUNK)],
                                      idx_b[b], sem_i[b]).wait()
                pltpu.make_async_copy(vals_hbm.at[pl.ds(base, CHUNK)],
                                      val_b[b], sem_v[b]).wait()
                pltpu.async_copy(val_b[b], acc.at[idx_b[b]], sem_s[b], add=True)

            pv = ci - NW

            @pl.when((pv >= 0) & (pv < N_CHUNKS))
            def _():
                pltpu.make_async_copy(val_b[nb], acc.at[idx_b[nb]],
                                      sem_s[nb]).wait()

            @pl.when(ci + NW < N_CHUNKS)
            def _():
                nbase = chunk_base(j + 1)
                pltpu.async_copy(col_hbm.at[pl.ds(nbase, CHUNK)], idx_b[nb], sem_i[nb])
                pltpu.async_copy(vals_hbm.at[pl.ds(nbase, CHUNK)], val_b[nb], sem_v[nb])

        def body(jj, carry):
            for b in range(NBUF):
                step(jj * NBUF + b, b)
            return carry

        lax.fori_loop(0, TOT_STEPS // NBUF, body, 0)
        j_last = TOT_STEPS - 1
        ci_last = wid + j_last * NW
        b_last = j_last % NBUF

        @pl.when(ci_last < N_CHUNKS)
        def _():
            pltpu.make_async_copy(val_b[b_last], acc.at[idx_b[b_last]],
                                  sem_s[b_last]).wait()

        plsc.subcore_barrier()
        pltpu.sync_copy(acc.at[pl.ds(rbase, ROWS_PER_TILE)],
                        out_hbm.at[cid, pl.ds(rbase, ROWS_PER_TILE)])

    return k(vals, col, zeros_nh)


def _count_sc(col, ones_ch, zeros_nh):
    @functools.partial(
        pl.kernel,
        mesh=_mesh(),
        out_type=jax.ShapeDtypeStruct((NC, NP, H), jnp.float32),
        scratch_types=[
            pltpu.VMEM_SHARED((NP, H), jnp.float32),
            pltpu.VMEM((CHUNK,), jnp.int32), pltpu.VMEM((CHUNK,), jnp.int32),
            pltpu.VMEM((CHUNK, H), jnp.float32),
        ] + [pltpu.SemaphoreType.DMA] * 4,
    )
    def k(col_hbm, ones_hbm, zeros_hbm, out_hbm, acc, idx0, idx1, ones_v,
          si0, si1, ss0, ss1):
        idx_b = [idx0, idx1]
        sem_i = [si0, si1]
        sem_s = [ss0, ss1]
        cid = lax.axis_index("c")
        sid = lax.axis_index("s")
        wid = sid * NC + cid
        rbase = pl.multiple_of(sid * ROWS_PER_TILE, 8)
        pltpu.sync_copy(zeros_hbm.at[pl.ds(rbase, ROWS_PER_TILE)],
                        acc.at[pl.ds(rbase, ROWS_PER_TILE)])
        pltpu.sync_copy(ones_hbm, ones_v)
        plsc.subcore_barrier()

        def chunk_base(j):
            return pl.multiple_of((wid + j * NW) * CHUNK, 8)

        pltpu.async_copy(col_hbm.at[pl.ds(chunk_base(0), CHUNK)], idx_b[0], sem_i[0])

        def step(j, b):
            ci = wid + j * NW
            nb = (b + 1) % NBUF

            @pl.when(ci < N_CHUNKS)
            def _():
                pltpu.make_async_copy(col_hbm.at[pl.ds(chunk_base(j), CHUNK)],
                                      idx_b[b], sem_i[b]).wait()
                pltpu.async_copy(ones_v, acc.at[idx_b[b]], sem_s[b], add=True)

            pv = ci - NW

            @pl.when((pv >= 0) & (pv < N_CHUNKS))
            def _():
                pltpu.make_async_copy(ones_v, acc.at[idx_b[nb]], sem_s[nb]).wait()

            @pl.when(ci + NW < N_CHUNKS)
            def _():
                pltpu.async_copy(col_hbm.at[pl.ds(chunk_base(j + 1), CHUNK)],
                                 idx_b[nb], sem_i[nb])

        def body(jj, carry):
            for b in range(NBUF):
                step(jj * NBUF + b, b)
            return carry

        lax.fori_loop(0, TOT_STEPS // NBUF, body, 0)
        j_last = TOT_STEPS - 1
        ci_last = wid + j_last * NW
        b_last = j_last % NBUF

        @pl.when(ci_last < N_CHUNKS)
        def _():
            pltpu.make_async_copy(ones_v, acc.at[idx_b[b_last]], sem_s[b_last]).wait()

        plsc.subcore_barrier()
        pltpu.sync_copy(acc.at[pl.ds(rbase, ROWS_PER_TILE)],
                        out_hbm.at[cid, pl.ds(rbase, ROWS_PER_TILE)])

    return k(col, ones_ch, zeros_nh)


def _ln(h, g, b):
    m = jnp.mean(h, axis=-1, keepdims=True)
    v = jnp.mean(jnp.square(h - m), axis=-1, keepdims=True)
    return (h - m) * lax.rsqrt(v + 1e-5) * g + b


def _gelu(h):
    return 0.5 * h * (1.0 + lax.erf(h * (1.0 / math.sqrt(2.0))))


E_B = 2000


def _edge_tc(xr, xc, ea, p):
    W1 = p["lin1"]["W"]
    w1a, w1b, w1c = W1[:H], W1[H:2 * H], W1[2 * H:]
    b1 = p["lin1"]["b"][None, :]
    w2 = p["lin2"]["W"]
    b2 = p["lin2"]["b"][None, :]
    g1, be1 = p["ln1_g"][None, :], p["ln1_b"][None, :]
    g2, be2 = p["ln2_g"][None, :], p["ln2_b"][None, :]

    def body(xr_r, xc_r, ea_r, w1a_r, w1b_r, w1c_r, b1_r, g1_r, be1_r,
             w2_r, b2_r, g2_r, be2_r, out_r):
        h = jnp.dot(xr_r[...], w1a_r[...], preferred_element_type=jnp.float32)
        h = h + jnp.dot(xc_r[...], w1b_r[...], preferred_element_type=jnp.float32)
        h = h + jnp.dot(ea_r[...], w1c_r[...], preferred_element_type=jnp.float32)
        h = h + b1_r[...]
        h = _gelu(_ln(h, g1_r[...], be1_r[...]))
        h2 = jnp.dot(h, w2_r[...], preferred_element_type=jnp.float32) + b2_r[...]
        h2 = _ln(h2, g2_r[...], be2_r[...])
        out_r[...] = ea_r[...] + h2

    eb = pl.BlockSpec((E_B, H), lambda i: (i, 0))
    hb = pl.BlockSpec((E_B, 2 * H), lambda i: (i, 0))
    full = lambda a: pl.BlockSpec(a.shape, lambda i: tuple(0 for _ in a.shape))
    return pl.pallas_call(
        body,
        grid=(E // E_B,),
        in_specs=[eb, eb, eb, full(w1a), full(w1b), full(w1c), full(b1),
                  full(g1), full(be1), full(w2), full(b2), full(g2), full(be2)],
        out_specs=eb,
        out_shape=jax.ShapeDtypeStruct((E, H), jnp.float32),
    )(xr, xc, ea, w1a, w1b, w1c, b1, g1, be1, w2, b2, g2, be2)


def _tokens_tc(x, p):
    te = p["token_embed"]
    a = p["attn_tok"]
    args = [x, te,
            a["q"]["W"], a["q"]["b"][None, :], a["k"]["W"], a["k"]["b"][None, :],
            a["v"]["W"], a["v"]["b"][None, :], a["o"]["W"], a["o"]["b"][None, :],
            p["ln_tok1_g"][None, :], p["ln_tok1_b"][None, :],
            p["ff1"]["W"], p["ff1"]["b"][None, :],
            p["ff2"]["W"], p["ff2"]["b"][None, :],
            p["ln_tok2_g"][None, :], p["ln_tok2_b"][None, :]]

    def body(x_r, te_r, wq, bq, wk, bk, wv, bv, wo, bo, g1, be1,
             wf1, bf1, wf2, bf2, g2, be2, out_r):
        xx = x_r[...]
        tok = te_r[...]
        q = jnp.dot(tok, wq[...], preferred_element_type=jnp.float32) + bq[...]
        kk = jnp.dot(xx, wk[...], preferred_element_type=jnp.float32) + bk[...]
        vv = jnp.dot(xx, wv[...], preferred_element_type=jnp.float32) + bv[...]
        outs = []
        scale = 1.0 / math.sqrt(float(HD))
        for hh in range(NHEADS):
            sl = slice(hh * HD, (hh + 1) * HD)
            logits = lax.dot_general(q[:, sl], kk[:, sl],
                                     (((1,), (1,)), ((), ()))) * scale
            m = jnp.max(logits, axis=-1, keepdims=True)
            ex = jnp.exp(logits - m)
            pr = ex / jnp.sum(ex, axis=-1, keepdims=True)
            outs.append(lax.dot_general(pr, vv[:, sl], (((1,), (0,)), ((), ()))))
        o = jnp.concatenate(outs, axis=1)
        tok = tok + jnp.dot(o, wo[...], preferred_element_type=jnp.float32) + bo[...]
        tok = _ln(tok, g1[...], be1[...])
        f = _gelu(jnp.dot(tok, wf1[...], preferred_element_type=jnp.float32) + bf1[...])
        tok = tok + jnp.dot(f, wf2[...], preferred_element_type=jnp.float32) + bf2[...]
        tok = _ln(tok, g2[...], be2[...])
        out_r[...] = tok

    full = lambda arr: pl.BlockSpec(arr.shape, lambda: tuple(0 for _ in arr.shape))
    return pl.pallas_call(
        body,
        in_specs=[full(a_) for a_ in args],
        out_shape=jax.ShapeDtypeStruct((2, H), jnp.float32),
    )(*args)


N_B = 2000


def _node_tc(x, parts, cnt_parts, tokens, p_attn, p_mlp):
    W1 = p_mlp["lin1"]["W"]
    w1a, w1b, w1c = W1[:H], W1[H:2 * H], W1[2 * H:]
    args = [x, parts[0], parts[1], cnt_parts[0], cnt_parts[1], tokens,
            p_attn["q"]["W"], p_attn["q"]["b"][None, :],
            p_attn["k"]["W"], p_attn["k"]["b"][None, :],
            p_attn["v"]["W"], p_attn["v"]["b"][None, :],
            p_attn["o"]["W"], p_attn["o"]["b"][None, :],
            w1a, w1b, w1c, p_mlp["lin1"]["b"][None, :],
            p_mlp["ln1_g"][None, :], p_mlp["ln1_b"][None, :],
            p_mlp["lin2"]["W"], p_mlp["lin2"]["b"][None, :],
            p_mlp["ln2_g"][None, :], p_mlp["ln2_b"][None, :]]

    def body(x_r, p0_r, p1_r, c0_r, c1_r, tok_r,
             wq, bq, wk, bk, wv, bv, wo, bo,
             w1a_r, w1b_r, w1c_r, b1_r, g1, be1, w2, b2, g2, be2, out_r):
        xx = x_r[...]
        cnt = c0_r[...][:, 0:1] + c1_r[...][:, 0:1]
        agg = (p0_r[...] + p1_r[...]) / jnp.maximum(cnt, 1.0)
        tok = tok_r[...]
        q = jnp.dot(xx, wq[...], preferred_element_type=jnp.float32) + bq[...]
        tk = jnp.dot(tok, wk[...], preferred_element_type=jnp.float32) + bk[...]
        tv = jnp.dot(tok, wv[...], preferred_element_type=jnp.float32) + bv[...]
        scale = 1.0 / math.sqrt(float(HD))
        outs = []
        for hh in range(NHEADS):
            sl = slice(hh * HD, (hh + 1) * HD)
            logits = lax.dot_general(q[:, sl], tk[:, sl],
                                     (((1,), (1,)), ((), ()))) * scale
            m = jnp.max(logits, axis=-1, keepdims=True)
            ex = jnp.exp(logits - m)
            pr = ex / jnp.sum(ex, axis=-1, keepdims=True)
            outs.append(lax.dot_general(pr, tv[:, sl], (((1,), (0,)), ((), ()))))
        ctx = jnp.dot(jnp.concatenate(outs, axis=1), wo[...],
                      preferred_element_type=jnp.float32) + bo[...]
        h = jnp.dot(xx, w1a_r[...], preferred_element_type=jnp.float32)
        h = h + jnp.dot(agg, w1b_r[...], preferred_element_type=jnp.float32)
        h = h + jnp.dot(ctx, w1c_r[...], preferred_element_type=jnp.float32)
        h = h + b1_r[...]
        h = _gelu(_ln(h, g1[...], be1[...]))
        h2 = jnp.dot(h, w2[...], preferred_element_type=jnp.float32) + b2[...]
        h2 = _ln(h2, g2[...], be2[...])
        out_r[...] = xx + h2

    nb = pl.BlockSpec((N_B, H), lambda i: (i, 0))
    cb = pl.BlockSpec((N_B, H), lambda i: (i, 0))
    full = lambda arr: pl.BlockSpec(arr.shape, lambda i: tuple(0 for _ in arr.shape))
    specs = [nb, nb, nb, cb, cb] + [full(a_) for a_ in args[5:]]
    return pl.pallas_call(
        body,
        grid=(N // N_B,),
        in_specs=specs,
        out_specs=nb,
        out_shape=jax.ShapeDtypeStruct((N, H), jnp.float32),
    )(*args)


def kernel(x, edge_index, edge_attr, params):
    row = edge_index[0]
    col = edge_index[1]
    zeros_nh = jnp.zeros((NP, H), jnp.float32)
    ones_ch = jnp.ones((CHUNK, H), jnp.float32)

    cnt_parts = _count_sc(col, ones_ch, zeros_nh)
    for i in range(2):
        xp = jnp.pad(x, ((0, NP - N), (0, 0)))
        xr, xc = _gather_sc(xp, row, col)
        edge_attr = _edge_tc(xr, xc, edge_attr, params["edge"][i])
        parts = _scatter_sc(edge_attr, col, zeros_nh)
        tokens = _tokens_tc(x, params["gtt"])
        x = _node_tc(x, parts, cnt_parts, tokens,
                     params["gtt"]["attn_node"], params["node"][i])
    return x, edge_attr


# speedup vs baseline: 1.0686x; 1.0503x over previous
import functools
import math

import jax
import jax.numpy as jnp
from jax import lax
from jax.experimental import pallas as pl
from jax.experimental.pallas import tpu as pltpu
from jax.experimental.pallas import tpu_sc as plsc

N = 10000
E = 320000
H = 128
NHEADS = 4
HD = H // NHEADS

NC = 2
NS = 16
NW = NC * NS
CHUNK = 128
N_CHUNKS = E // CHUNK
CH_PER_W = -(-N_CHUNKS // NW)
NP = 10240
ROWS_PER_TILE = NP // NS


def _mesh():
    return plsc.VectorSubcoreMesh(core_axis_name="c", subcore_axis_name="s")


NBUF = 2
TOT_STEPS = -(-CH_PER_W // NBUF) * NBUF
NBUFG = 3
TOTG = -(-CH_PER_W // NBUFG) * NBUFG


def _gather_sc(x, row, col):
    @functools.partial(
        pl.kernel,
        mesh=_mesh(),
        out_type=(jax.ShapeDtypeStruct((E, H), jnp.float32),
                  jax.ShapeDtypeStruct((E, H), jnp.float32)),
        scratch_types=[pltpu.VMEM((CHUNK,), jnp.int32)] * 6
                      + [pltpu.VMEM((CHUNK, H), jnp.float32)] * 6
                      + [pltpu.SemaphoreType.DMA] * 18,
    )
    def k(x_hbm, row_hbm, col_hbm, xr_hbm, xc_hbm,
          ir0, ir1, ir2, ic0, ic1, ic2, rr0, rr1, rr2, rc0, rc1, rc2,
          sir0, sir1, sir2, sic0, sic1, sic2, sgr0, sgr1, sgr2,
          sgc0, sgc1, sgc2, swr0, swr1, swr2, swc0, swc1, swc2):
        idx_r = [ir0, ir1, ir2]
        idx_c = [ic0, ic1, ic2]
        rows_r = [rr0, rr1, rr2]
        rows_c = [rc0, rc1, rc2]
        sem_ir = [sir0, sir1, sir2]
        sem_ic = [sic0, sic1, sic2]
        sem_gr = [sgr0, sgr1, sgr2]
        sem_gc = [sgc0, sgc1, sgc2]
        sem_wr = [swr0, swr1, swr2]
        sem_wc = [swc0, swc1, swc2]
        cid = lax.axis_index("c")
        sid = lax.axis_index("s")
        wid = sid * NC + cid

        def chunk_base(j):
            return pl.multiple_of((wid + j * NW) * CHUNK, 8)

        b0 = chunk_base(0)
        pltpu.async_copy(row_hbm.at[pl.ds(b0, CHUNK)], idx_r[0], sem_ir[0])
        pltpu.async_copy(col_hbm.at[pl.ds(b0, CHUNK)], idx_c[0], sem_ic[0])

        def step(j, b):
            ci = wid + j * NW
            valid = ci < N_CHUNKS
            nb = (b + 1) % NBUFG

            @pl.when(valid)
            def _():
                pltpu.make_async_copy(row_hbm.at[pl.ds(chunk_base(j), CHUNK)],
                                      idx_r[b], sem_ir[b]).wait()
                pltpu.make_async_copy(col_hbm.at[pl.ds(chunk_base(j), CHUNK)],
                                      idx_c[b], sem_ic[b]).wait()

            @pl.when(ci + NW < N_CHUNKS)
            def _():
                nbase = chunk_base(j + 1)
                pltpu.async_copy(row_hbm.at[pl.ds(nbase, CHUNK)], idx_r[nb], sem_ir[nb])
                pltpu.async_copy(col_hbm.at[pl.ds(nbase, CHUNK)], idx_c[nb], sem_ic[nb])

            pv = ci - NBUFG * NW

            @pl.when((pv >= 0) & (pv < N_CHUNKS))
            def _():
                pbase = chunk_base(j - NBUFG)
                pltpu.make_async_copy(rows_r[b], xr_hbm.at[pl.ds(pbase, CHUNK)],
                                      sem_wr[b]).wait()
                pltpu.make_async_copy(rows_c[b], xc_hbm.at[pl.ds(pbase, CHUNK)],
                                      sem_wc[b]).wait()

            @pl.when(valid)
            def _():
                base = chunk_base(j)
                pltpu.async_copy(x_hbm.at[idx_r[b]], rows_r[b], sem_gr[b])
                pltpu.async_copy(x_hbm.at[idx_c[b]], rows_c[b], sem_gc[b])
                pltpu.make_async_copy(x_hbm.at[idx_r[b]], rows_r[b], sem_gr[b]).wait()
                pltpu.make_async_copy(x_hbm.at[idx_c[b]], rows_c[b], sem_gc[b]).wait()
                pltpu.async_copy(rows_r[b], xr_hbm.at[pl.ds(base, CHUNK)], sem_wr[b])
                pltpu.async_copy(rows_c[b], xc_hbm.at[pl.ds(base, CHUNK)], sem_wc[b])

        def body(jj, carry):
            for b in range(NBUFG):
                step(jj * NBUFG + b, b)
            return carry

        lax.fori_loop(0, TOTG // NBUFG, body, 0)
        for b in range(NBUFG):
            j = TOTG - NBUFG + b
            ci = wid + j * NW

            @pl.when(ci < N_CHUNKS)
            def _():
                base = chunk_base(j)
                pltpu.make_async_copy(rows_r[b], xr_hbm.at[pl.ds(base, CHUNK)],
                                      sem_wr[b]).wait()
                pltpu.make_async_copy(rows_c[b], xc_hbm.at[pl.ds(base, CHUNK)],
                                      sem_wc[b]).wait()

    return k(x, row, col)


def _scatter_sc(vals, col, zeros_nh):
    @functools.partial(
        pl.kernel,
        mesh=_mesh(),
        out_type=jax.ShapeDtypeStruct((NC, NP, H), jnp.float32),
        scratch_types=[
            pltpu.VMEM_SHARED((NP, H), jnp.float32),
            pltpu.VMEM((CHUNK,), jnp.int32), pltpu.VMEM((CHUNK,), jnp.int32),
            pltpu.VMEM((CHUNK, H), jnp.float32), pltpu.VMEM((CHUNK, H), jnp.float32),
        ] + [pltpu.SemaphoreType.DMA] * 6,
    )
    def k(vals_hbm, col_hbm, zeros_hbm, out_hbm, acc, idx0, idx1, val0, val1,
          si0, si1, sv0, sv1, ss0, ss1):
        idx_b = [idx0, idx1]
        val_b = [val0, val1]
        sem_i = [si0, si1]
        sem_v = [sv0, sv1]
        sem_s = [ss0, ss1]
        cid = lax.axis_index("c")
        sid = lax.axis_index("s")
        wid = sid * NC + cid
        rbase = pl.multiple_of(sid * ROWS_PER_TILE, 8)
        pltpu.sync_copy(zeros_hbm.at[pl.ds(rbase, ROWS_PER_TILE)],
                        acc.at[pl.ds(rbase, ROWS_PER_TILE)])
        plsc.subcore_barrier()

        def chunk_base(j):
            return pl.multiple_of((wid + j * NW) * CHUNK, 8)

        b0 = chunk_base(0)
        pltpu.async_copy(col_hbm.at[pl.ds(b0, CHUNK)], idx_b[0], sem_i[0])
        pltpu.async_copy(vals_hbm.at[pl.ds(b0, CHUNK)], val_b[0], sem_v[0])

        def step(j, b):
            ci = wid + j * NW
            valid = ci < N_CHUNKS
            nb = (b + 1) % NBUF

            @pl.when(valid)
            def _():
                base = chunk_base(j)
                pltpu.make_async_copy(col_hbm.at[pl.ds(base, CHUNK)],
                                      idx_b[b], sem_i[b]).wait()
                pltpu.make_async_copy(vals_hbm.at[pl.ds(base, CHUNK)],
                                      val_b[b], sem_v[b]).wait()
                pltpu.async_copy(val_b[b], acc.at[idx_b[b]], sem_s[b], add=True)

            pv = ci - NW

            @pl.when((pv >= 0) & (pv < N_CHUNKS))
            def _():
                pltpu.make_async_copy(val_b[nb], acc.at[idx_b[nb]],
                                      sem_s[nb]).wait()

            @pl.when(ci + NW < N_CHUNKS)
            def _():
                nbase = chunk_base(j + 1)
                pltpu.async_copy(col_hbm.at[pl.ds(nbase, CHUNK)], idx_b[nb], sem_i[nb])
                pltpu.async_copy(vals_hbm.at[pl.ds(nbase, CHUNK)], val_b[nb], sem_v[nb])

        def body(jj, carry):
            for b in range(NBUF):
                step(jj * NBUF + b, b)
            return carry

        lax.fori_loop(0, TOT_STEPS // NBUF, body, 0)
        j_last = TOT_STEPS - 1
        ci_last = wid + j_last * NW
        b_last = j_last % NBUF

        @pl.when(ci_last < N_CHUNKS)
        def _():
            pltpu.make_async_copy(val_b[b_last], acc.at[idx_b[b_last]],
                                  sem_s[b_last]).wait()

        plsc.subcore_barrier()
        pltpu.sync_copy(acc.at[pl.ds(rbase, ROWS_PER_TILE)],
                        out_hbm.at[cid, pl.ds(rbase, ROWS_PER_TILE)])

    return k(vals, col, zeros_nh)


def _count_sc(col, ones_ch, zeros_nh):
    @functools.partial(
        pl.kernel,
        mesh=_mesh(),
        out_type=jax.ShapeDtypeStruct((NC, NP, H), jnp.float32),
        scratch_types=[
            pltpu.VMEM_SHARED((NP, H), jnp.float32),
            pltpu.VMEM((CHUNK,), jnp.int32), pltpu.VMEM((CHUNK,), jnp.int32),
            pltpu.VMEM((CHUNK, H), jnp.float32),
        ] + [pltpu.SemaphoreType.DMA] * 4,
    )
    def k(col_hbm, ones_hbm, zeros_hbm, out_hbm, acc, idx0, idx1, ones_v,
          si0, si1, ss0, ss1):
        idx_b = [idx0, idx1]
        sem_i = [si0, si1]
        sem_s = [ss0, ss1]
        cid = lax.axis_index("c")
        sid = lax.axis_index("s")
        wid = sid * NC + cid
        rbase = pl.multiple_of(sid * ROWS_PER_TILE, 8)
        pltpu.sync_copy(zeros_hbm.at[pl.ds(rbase, ROWS_PER_TILE)],
                        acc.at[pl.ds(rbase, ROWS_PER_TILE)])
        pltpu.sync_copy(ones_hbm, ones_v)
        plsc.subcore_barrier()

        def chunk_base(j):
            return pl.multiple_of((wid + j * NW) * CHUNK, 8)

        pltpu.async_copy(col_hbm.at[pl.ds(chunk_base(0), CHUNK)], idx_b[0], sem_i[0])

        def step(j, b):
            ci = wid + j * NW
            nb = (b + 1) % NBUF

            @pl.when(ci < N_CHUNKS)
            def _():
                pltpu.make_async_copy(col_hbm.at[pl.ds(chunk_base(j), CHUNK)],
                                      idx_b[b], sem_i[b]).wait()
                pltpu.async_copy(ones_v, acc.at[idx_b[b]], sem_s[b], add=True)

            pv = ci - NW

            @pl.when((pv >= 0) & (pv < N_CHUNKS))
            def _():
                pltpu.make_async_copy(ones_v, acc.at[idx_b[nb]], sem_s[nb]).wait()

            @pl.when(ci + NW < N_CHUNKS)
            def _():
                pltpu.async_copy(col_hbm.at[pl.ds(chunk_base(j + 1), CHUNK)],
                                 idx_b[nb], sem_i[nb])

        def body(jj, carry):
            for b in range(NBUF):
                step(jj * NBUF + b, b)
            return carry

        lax.fori_loop(0, TOT_STEPS // NBUF, body, 0)
        j_last = TOT_STEPS - 1
        ci_last = wid + j_last * NW
        b_last = j_last % NBUF

        @pl.when(ci_last < N_CHUNKS)
        def _():
            pltpu.make_async_copy(ones_v, acc.at[idx_b[b_last]], sem_s[b_last]).wait()

        plsc.subcore_barrier()
        pltpu.sync_copy(acc.at[pl.ds(rbase, ROWS_PER_TILE)],
                        out_hbm.at[cid, pl.ds(rbase, ROWS_PER_TILE)])

    return k(col, ones_ch, zeros_nh)


def _ln(h, g, b):
    m = jnp.mean(h, axis=-1, keepdims=True)
    v = jnp.mean(jnp.square(h - m), axis=-1, keepdims=True)
    return (h - m) * lax.rsqrt(v + 1e-5) * g + b


def _gelu(h):
    return 0.5 * h * (1.0 + lax.erf(h * (1.0 / math.sqrt(2.0))))


E_B = 4000


def _edge_tc(xr, xc, ea, p):
    W1 = p["lin1"]["W"]
    w1a, w1b, w1c = W1[:H], W1[H:2 * H], W1[2 * H:]
    b1 = p["lin1"]["b"][None, :]
    w2 = p["lin2"]["W"]
    b2 = p["lin2"]["b"][None, :]
    g1, be1 = p["ln1_g"][None, :], p["ln1_b"][None, :]
    g2, be2 = p["ln2_g"][None, :], p["ln2_b"][None, :]

    def body(xr_r, xc_r, ea_r, w1a_r, w1b_r, w1c_r,
             b1_r, g1_r, be1_r, w2_r, b2_r, g2_r, be2_r, out_r):
        f32 = jnp.float32
        h = jnp.dot(xr_r[...], w1a_r[...], preferred_element_type=f32)
        h = h + jnp.dot(xc_r[...], w1b_r[...], preferred_element_type=f32)
        h = h + jnp.dot(ea_r[...], w1c_r[...], preferred_element_type=f32)
        h = h + b1_r[...]
        h = _gelu(_ln(h, g1_r[...], be1_r[...]))
        h2 = jnp.dot(h, w2_r[...], preferred_element_type=jnp.float32) + b2_r[...]
        h2 = _ln(h2, g2_r[...], be2_r[...])
        out_r[...] = ea_r[...] + h2

    eb = pl.BlockSpec((E_B, H), lambda i: (i, 0))
    pb = pl.BlockSpec((E_B, H), lambda i: (i, 0))
    full = lambda a: pl.BlockSpec(a.shape, lambda i: tuple(0 for _ in a.shape))
    return pl.pallas_call(
        body,
        grid=(E // E_B,),
        in_specs=[pb, pb, eb, full(w1a), full(w1b), full(w1c), full(b1),
                  full(g1), full(be1), full(w2), full(b2), full(g2), full(be2)],
        out_specs=eb,
        out_shape=jax.ShapeDtypeStruct((E, H), jnp.float32),
    )(xr, xc, ea, w1a, w1b, w1c, b1, g1, be1, w2, b2, g2, be2)


def _tokens_tc(x, p):
    te = p["token_embed"]
    a = p["attn_tok"]
    args = [x, te,
            a["q"]["W"], a["q"]["b"][None, :], a["k"]["W"], a["k"]["b"][None, :],
            a["v"]["W"], a["v"]["b"][None, :], a["o"]["W"], a["o"]["b"][None, :],
            p["ln_tok1_g"][None, :], p["ln_tok1_b"][None, :],
            p["ff1"]["W"], p["ff1"]["b"][None, :],
            p["ff2"]["W"], p["ff2"]["b"][None, :],
            p["ln_tok2_g"][None, :], p["ln_tok2_b"][None, :]]

    def body(x_r, te_r, wq, bq, wk, bk, wv, bv, wo, bo, g1, be1,
             wf1, bf1, wf2, bf2, g2, be2, out_r):
        xx = x_r[...]
        tok = te_r[...]
        q = jnp.dot(tok, wq[...], preferred_element_type=jnp.float32) + bq[...]
        kk = jnp.dot(xx, wk[...], preferred_element_type=jnp.float32) + bk[...]
        vv = jnp.dot(xx, wv[...], preferred_element_type=jnp.float32) + bv[...]
        outs = []
        scale = 1.0 / math.sqrt(float(HD))
        for hh in range(NHEADS):
            sl = slice(hh * HD, (hh + 1) * HD)
            logits = lax.dot_general(q[:, sl], kk[:, sl],
                                     (((1,), (1,)), ((), ()))) * scale
            m = jnp.max(logits, axis=-1, keepdims=True)
            ex = jnp.exp(logits - m)
            pr = ex / jnp.sum(ex, axis=-1, keepdims=True)
            outs.append(lax.dot_general(pr, vv[:, sl], (((1,), (0,)), ((), ()))))
        o = jnp.concatenate(outs, axis=1)
        tok = tok + jnp.dot(o, wo[...], preferred_element_type=jnp.float32) + bo[...]
        tok = _ln(tok, g1[...], be1[...])
        f = _gelu(jnp.dot(tok, wf1[...], preferred_element_type=jnp.float32) + bf1[...])
        tok = tok + jnp.dot(f, wf2[...], preferred_element_type=jnp.float32) + bf2[...]
        tok = _ln(tok, g2[...], be2[...])
        out_r[...] = tok

    full = lambda arr: pl.BlockSpec(arr.shape, lambda: tuple(0 for _ in arr.shape))
    return pl.pallas_call(
        body,
        in_specs=[full(a_) for a_ in args],
        out_shape=jax.ShapeDtypeStruct((2, H), jnp.float32),
    )(*args)


N_B = 2000


def _node_tc(x, parts, cnt_parts, tokens, p_attn, p_mlp):
    W1 = p_mlp["lin1"]["W"]
    w1a, w1b, w1c = W1[:H], W1[H:2 * H], W1[2 * H:]
    args = [x, parts[0], parts[1], cnt_parts[0], cnt_parts[1], tokens,
            p_attn["q"]["W"], p_attn["q"]["b"][None, :],
            p_attn["k"]["W"], p_attn["k"]["b"][None, :],
            p_attn["v"]["W"], p_attn["v"]["b"][None, :],
            p_attn["o"]["W"], p_attn["o"]["b"][None, :],
            w1a, w1b, w1c, p_mlp["lin1"]["b"][None, :],
            p_mlp["ln1_g"][None, :], p_mlp["ln1_b"][None, :],
            p_mlp["lin2"]["W"], p_mlp["lin2"]["b"][None, :],
            p_mlp["ln2_g"][None, :], p_mlp["ln2_b"][None, :]]

    def body(x_r, p0_r, p1_r, c0_r, c1_r, tok_r,
             wq, bq, wk, bk, wv, bv, wo, bo,
             w1a_r, w1b_r, w1c_r, b1_r, g1, be1, w2, b2, g2, be2, out_r):
        xx = x_r[...]
        cnt = c0_r[...][:, 0:1] + c1_r[...][:, 0:1]
        agg = (p0_r[...] + p1_r[...]) / jnp.maximum(cnt, 1.0)
        tok = tok_r[...]
        q = jnp.dot(xx, wq[...], preferred_element_type=jnp.float32) + bq[...]
        tk = jnp.dot(tok, wk[...], preferred_element_type=jnp.float32) + bk[...]
        tv = jnp.dot(tok, wv[...], preferred_element_type=jnp.float32) + bv[...]
        scale = 1.0 / math.sqrt(float(HD))
        outs = []
        for hh in range(NHEADS):
            sl = slice(hh * HD, (hh + 1) * HD)
            logits = lax.dot_general(q[:, sl], tk[:, sl],
                                     (((1,), (1,)), ((), ()))) * scale
            m = jnp.max(logits, axis=-1, keepdims=True)
            ex = jnp.exp(logits - m)
            pr = ex / jnp.sum(ex, axis=-1, keepdims=True)
            outs.append(lax.dot_general(pr, tv[:, sl], (((1,), (0,)), ((), ()))))
        ctx = jnp.dot(jnp.concatenate(outs, axis=1), wo[...],
                      preferred_element_type=jnp.float32) + bo[...]
        h = jnp.dot(xx, w1a_r[...], preferred_element_type=jnp.float32)
        h = h + jnp.dot(agg, w1b_r[...], preferred_element_type=jnp.float32)
        h = h + jnp.dot(ctx, w1c_r[...], preferred_element_type=jnp.float32)
        h = h + b1_r[...]
        h = _gelu(_ln(h, g1[...], be1[...]))
        h2 = jnp.dot(h, w2[...], preferred_element_type=jnp.float32) + b2[...]
        h2 = _ln(h2, g2[...], be2[...])
        out_r[...] = xx + h2

    nb = pl.BlockSpec((N_B, H), lambda i: (i, 0))
    cb = pl.BlockSpec((N_B, H), lambda i: (i, 0))
    full = lambda arr: pl.BlockSpec(arr.shape, lambda i: tuple(0 for _ in arr.shape))
    specs = [nb, nb, nb, cb, cb] + [full(a_) for a_ in args[5:]]
    return pl.pallas_call(
        body,
        grid=(N // N_B,),
        in_specs=specs,
        out_specs=nb,
        out_shape=jax.ShapeDtypeStruct((N, H), jnp.float32),
    )(*args)


def kernel(x, edge_index, edge_attr, params):
    row = edge_index[0]
    col = edge_index[1]
    zeros_nh = jnp.zeros((NP, H), jnp.float32)
    ones_ch = jnp.ones((CHUNK, H), jnp.float32)

    cnt_parts = _count_sc(col, ones_ch, zeros_nh)
    for i in range(2):
        xp = jnp.pad(x, ((0, NP - N), (0, 0)))
        xr, xc = _gather_sc(xp, row, col)
        edge_attr = _edge_tc(xr, xc, edge_attr, params["edge"][i])
        parts = _scatter_sc(edge_attr, col, zeros_nh)
        tokens = _tokens_tc(x, params["gtt"])
        x = _node_tc(x, parts, cnt_parts, tokens,
                     params["gtt"]["attn_node"], params["node"][i])
    return x, edge_attr


# speedup vs baseline: 1.0740x; 1.0050x over previous
import functools
import math

import jax
import jax.numpy as jnp
from jax import lax
from jax.experimental import pallas as pl
from jax.experimental.pallas import tpu as pltpu
from jax.experimental.pallas import tpu_sc as plsc

N = 10000
E = 320000
H = 128
NHEADS = 4
HD = H // NHEADS

NC = 2
NS = 16
NW = NC * NS
CHUNK = 128
N_CHUNKS = E // CHUNK
CH_PER_W = -(-N_CHUNKS // NW)
NP = 10240
ROWS_PER_TILE = NP // NS


def _mesh():
    return plsc.VectorSubcoreMesh(core_axis_name="c", subcore_axis_name="s")


NBUF = 2
TOT_STEPS = -(-CH_PER_W // NBUF) * NBUF
NBUFG = 3
TOTG = -(-CH_PER_W // NBUFG) * NBUFG
NPS = 10112
RPTS = NPS // NS


def _gather_sc(x, row, col):
    @functools.partial(
        pl.kernel,
        mesh=_mesh(),
        out_type=(jax.ShapeDtypeStruct((E, H), jnp.float32),
                  jax.ShapeDtypeStruct((E, H), jnp.float32)),
        scratch_types=[pltpu.VMEM((CHUNK,), jnp.int32)] * 6
                      + [pltpu.VMEM((CHUNK, H), jnp.float32)] * 6
                      + [pltpu.SemaphoreType.DMA] * 18,
    )
    def k(x_hbm, row_hbm, col_hbm, xr_hbm, xc_hbm,
          ir0, ir1, ir2, ic0, ic1, ic2, rr0, rr1, rr2, rc0, rc1, rc2,
          sir0, sir1, sir2, sic0, sic1, sic2, sgr0, sgr1, sgr2,
          sgc0, sgc1, sgc2, swr0, swr1, swr2, swc0, swc1, swc2):
        idx_r = [ir0, ir1, ir2]
        idx_c = [ic0, ic1, ic2]
        rows_r = [rr0, rr1, rr2]
        rows_c = [rc0, rc1, rc2]
        sem_ir = [sir0, sir1, sir2]
        sem_ic = [sic0, sic1, sic2]
        sem_gr = [sgr0, sgr1, sgr2]
        sem_gc = [sgc0, sgc1, sgc2]
        sem_wr = [swr0, swr1, swr2]
        sem_wc = [swc0, swc1, swc2]
        cid = lax.axis_index("c")
        sid = lax.axis_index("s")
        wid = sid * NC + cid

        def chunk_base(j):
            return pl.multiple_of((wid + j * NW) * CHUNK, 8)

        b0 = chunk_base(0)
        pltpu.async_copy(row_hbm.at[pl.ds(b0, CHUNK)], idx_r[0], sem_ir[0])
        pltpu.async_copy(col_hbm.at[pl.ds(b0, CHUNK)], idx_c[0], sem_ic[0])

        def step(j, b):
            ci = wid + j * NW
            valid = ci < N_CHUNKS
            nb = (b + 1) % NBUFG

            @pl.when(valid)
            def _():
                pltpu.make_async_copy(row_hbm.at[pl.ds(chunk_base(j), CHUNK)],
                                      idx_r[b], sem_ir[b]).wait()
                pltpu.make_async_copy(col_hbm.at[pl.ds(chunk_base(j), CHUNK)],
                                      idx_c[b], sem_ic[b]).wait()

            @pl.when(ci + NW < N_CHUNKS)
            def _():
                nbase = chunk_base(j + 1)
                pltpu.async_copy(row_hbm.at[pl.ds(nbase, CHUNK)], idx_r[nb], sem_ir[nb])
                pltpu.async_copy(col_hbm.at[pl.ds(nbase, CHUNK)], idx_c[nb], sem_ic[nb])

            pv = ci - NBUFG * NW

            @pl.when((pv >= 0) & (pv < N_CHUNKS))
            def _():
                pbase = chunk_base(j - NBUFG)
                pltpu.make_async_copy(rows_r[b], xr_hbm.at[pl.ds(pbase, CHUNK)],
                                      sem_wr[b]).wait()
                pltpu.make_async_copy(rows_c[b], xc_hbm.at[pl.ds(pbase, CHUNK)],
                                      sem_wc[b]).wait()

            @pl.when(valid)
            def _():
                base = chunk_base(j)
                pltpu.async_copy(x_hbm.at[idx_r[b]], rows_r[b], sem_gr[b])
                pltpu.async_copy(x_hbm.at[idx_c[b]], rows_c[b], sem_gc[b])
                pltpu.make_async_copy(x_hbm.at[idx_r[b]], rows_r[b], sem_gr[b]).wait()
                pltpu.make_async_copy(x_hbm.at[idx_c[b]], rows_c[b], sem_gc[b]).wait()
                pltpu.async_copy(rows_r[b], xr_hbm.at[pl.ds(base, CHUNK)], sem_wr[b])
                pltpu.async_copy(rows_c[b], xc_hbm.at[pl.ds(base, CHUNK)], sem_wc[b])

        def body(jj, carry):
            for b in range(NBUFG):
                step(jj * NBUFG + b, b)
            return carry

        lax.fori_loop(0, TOTG // NBUFG, body, 0)
        for b in range(NBUFG):
            j = TOTG - NBUFG + b
            ci = wid + j * NW

            @pl.when(ci < N_CHUNKS)
            def _():
                base = chunk_base(j)
                pltpu.make_async_copy(rows_r[b], xr_hbm.at[pl.ds(base, CHUNK)],
                                      sem_wr[b]).wait()
                pltpu.make_async_copy(rows_c[b], xc_hbm.at[pl.ds(base, CHUNK)],
                                      sem_wc[b]).wait()

    return k(x, row, col)


def _scatter_sc(vals, col, zeros_nh):
    @functools.partial(
        pl.kernel,
        mesh=_mesh(),
        out_type=jax.ShapeDtypeStruct((NC, NPS, H), jnp.float32),
        scratch_types=[pltpu.VMEM_SHARED((NPS, H), jnp.float32)]
                      + [pltpu.VMEM((CHUNK,), jnp.int32)] * 3
                      + [pltpu.VMEM((CHUNK, H), jnp.float32)] * 3
                      + [pltpu.SemaphoreType.DMA] * 9,
    )
    def k(vals_hbm, col_hbm, zeros_hbm, out_hbm, acc, idx0, idx1, idx2,
          val0, val1, val2, si0, si1, si2, sv0, sv1, sv2, ss0, ss1, ss2):
        idx_b = [idx0, idx1, idx2]
        val_b = [val0, val1, val2]
        sem_i = [si0, si1, si2]
        sem_v = [sv0, sv1, sv2]
        sem_s = [ss0, ss1, ss2]
        cid = lax.axis_index("c")
        sid = lax.axis_index("s")
        wid = sid * NC + cid
        rbase = pl.multiple_of(sid * RPTS, 8)
        pltpu.sync_copy(zeros_hbm.at[pl.ds(rbase, RPTS)],
                        acc.at[pl.ds(rbase, RPTS)])
        plsc.subcore_barrier()

        def chunk_base(j):
            return pl.multiple_of((wid + j * NW) * CHUNK, 8)

        b0 = chunk_base(0)
        pltpu.async_copy(col_hbm.at[pl.ds(b0, CHUNK)], idx_b[0], sem_i[0])
        pltpu.async_copy(vals_hbm.at[pl.ds(b0, CHUNK)], val_b[0], sem_v[0])

        def step(j, b):
            ci = wid + j * NW
            valid = ci < N_CHUNKS
            nb = (b + 1) % NBUFG

            @pl.when(valid)
            def _():
                base = chunk_base(j)
                pltpu.make_async_copy(col_hbm.at[pl.ds(base, CHUNK)],
                                      idx_b[b], sem_i[b]).wait()
                pltpu.make_async_copy(vals_hbm.at[pl.ds(base, CHUNK)],
                                      val_b[b], sem_v[b]).wait()
                pltpu.async_copy(val_b[b], acc.at[idx_b[b]], sem_s[b], add=True)

            pv = ci - (NBUFG - 1) * NW

            @pl.when((pv >= 0) & (pv < N_CHUNKS))
            def _():
                pltpu.make_async_copy(val_b[nb], acc.at[idx_b[nb]],
                                      sem_s[nb]).wait()

            @pl.when(ci + NW < N_CHUNKS)
            def _():
                nbase = chunk_base(j + 1)
                pltpu.async_copy(col_hbm.at[pl.ds(nbase, CHUNK)], idx_b[nb], sem_i[nb])
                pltpu.async_copy(vals_hbm.at[pl.ds(nbase, CHUNK)], val_b[nb], sem_v[nb])

        def body(jj, carry):
            for b in range(NBUFG):
                step(jj * NBUFG + b, b)
            return carry

        lax.fori_loop(0, TOTG // NBUFG, body, 0)
        for t in (TOTG - 2, TOTG - 1):
            ci_t = wid + t * NW

            @pl.when(ci_t < N_CHUNKS)
            def _():
                bt = t % NBUFG
                pltpu.make_async_copy(val_b[bt], acc.at[idx_b[bt]],
                                      sem_s[bt]).wait()

        plsc.subcore_barrier()
        pltpu.sync_copy(acc.at[pl.ds(rbase, RPTS)],
                        out_hbm.at[cid, pl.ds(rbase, RPTS)])

    return k(vals, col, zeros_nh)


def _count_sc(col, ones_ch, zeros_nh):
    @functools.partial(
        pl.kernel,
        mesh=_mesh(),
        out_type=jax.ShapeDtypeStruct((NC, NP, H), jnp.float32),
        scratch_types=[
            pltpu.VMEM_SHARED((NP, H), jnp.float32),
            pltpu.VMEM((CHUNK,), jnp.int32), pltpu.VMEM((CHUNK,), jnp.int32),
            pltpu.VMEM((CHUNK, H), jnp.float32),
        ] + [pltpu.SemaphoreType.DMA] * 4,
    )
    def k(col_hbm, ones_hbm, zeros_hbm, out_hbm, acc, idx0, idx1, ones_v,
          si0, si1, ss0, ss1):
        idx_b = [idx0, idx1]
        sem_i = [si0, si1]
        sem_s = [ss0, ss1]
        cid = lax.axis_index("c")
        sid = lax.axis_index("s")
        wid = sid * NC + cid
        rbase = pl.multiple_of(sid * ROWS_PER_TILE, 8)
        pltpu.sync_copy(zeros_hbm.at[pl.ds(rbase, ROWS_PER_TILE)],
                        acc.at[pl.ds(rbase, ROWS_PER_TILE)])
        pltpu.sync_copy(ones_hbm, ones_v)
        plsc.subcore_barrier()

        def chunk_base(j):
            return pl.multiple_of((wid + j * NW) * CHUNK, 8)

        pltpu.async_copy(col_hbm.at[pl.ds(chunk_base(0), CHUNK)], idx_b[0], sem_i[0])

        def step(j, b):
            ci = wid + j * NW
            nb = (b + 1) % NBUF

            @pl.when(ci < N_CHUNKS)
            def _():
                pltpu.make_async_copy(col_hbm.at[pl.ds(chunk_base(j), CHUNK)],
                                      idx_b[b], sem_i[b]).wait()
                pltpu.async_copy(ones_v, acc.at[idx_b[b]], sem_s[b], add=True)

            pv = ci - NW

            @pl.when((pv >= 0) & (pv < N_CHUNKS))
            def _():
                pltpu.make_async_copy(ones_v, acc.at[idx_b[nb]], sem_s[nb]).wait()

            @pl.when(ci + NW < N_CHUNKS)
            def _():
                pltpu.async_copy(col_hbm.at[pl.ds(chunk_base(j + 1), CHUNK)],
                                 idx_b[nb], sem_i[nb])

        def body(jj, carry):
            for b in range(NBUF):
                step(jj * NBUF + b, b)
            return carry

        lax.fori_loop(0, TOT_STEPS // NBUF, body, 0)
        j_last = TOT_STEPS - 1
        ci_last = wid + j_last * NW
        b_last = j_last % NBUF

        @pl.when(ci_last < N_CHUNKS)
        def _():
            pltpu.make_async_copy(ones_v, acc.at[idx_b[b_last]], sem_s[b_last]).wait()

        plsc.subcore_barrier()
        pltpu.sync_copy(acc.at[pl.ds(rbase, ROWS_PER_TILE)],
                        out_hbm.at[cid, pl.ds(rbase, ROWS_PER_TILE)])

    return k(col, ones_ch, zeros_nh)


def _ln(h, g, b):
    m = jnp.mean(h, axis=-1, keepdims=True)
    v = jnp.mean(jnp.square(h - m), axis=-1, keepdims=True)
    return (h - m) * lax.rsqrt(v + 1e-5) * g + b


def _gelu(h):
    return 0.5 * h * (1.0 + lax.erf(h * (1.0 / math.sqrt(2.0))))


E_B = 5000


def _edge_tc(xr, xc, ea, p):
    W1 = p["lin1"]["W"]
    w1a, w1b, w1c = W1[:H], W1[H:2 * H], W1[2 * H:]
    b1 = p["lin1"]["b"][None, :]
    w2 = p["lin2"]["W"]
    b2 = p["lin2"]["b"][None, :]
    g1, be1 = p["ln1_g"][None, :], p["ln1_b"][None, :]
    g2, be2 = p["ln2_g"][None, :], p["ln2_b"][None, :]

    def body(xr_r, xc_r, ea_r, w1a_r, w1b_r, w1c_r,
             b1_r, g1_r, be1_r, w2_r, b2_r, g2_r, be2_r, out_r):
        f32 = jnp.float32
        h = jnp.dot(xr_r[...], w1a_r[...], preferred_element_type=f32)
        h = h + jnp.dot(xc_r[...], w1b_r[...], preferred_element_type=f32)
        h = h + jnp.dot(ea_r[...], w1c_r[...], preferred_element_type=f32)
        h = h + b1_r[...]
        h = _gelu(_ln(h, g1_r[...], be1_r[...]))
        h2 = jnp.dot(h, w2_r[...], preferred_element_type=jnp.float32) + b2_r[...]
        h2 = _ln(h2, g2_r[...], be2_r[...])
        out_r[...] = ea_r[...] + h2

    eb = pl.BlockSpec((E_B, H), lambda i: (i, 0))
    pb = pl.BlockSpec((E_B, H), lambda i: (i, 0))
    full = lambda a: pl.BlockSpec(a.shape, lambda i: tuple(0 for _ in a.shape))
    return pl.pallas_call(
        body,
        grid=(E // E_B,),
        in_specs=[pb, pb, eb, full(w1a), full(w1b), full(w1c), full(b1),
                  full(g1), full(be1), full(w2), full(b2), full(g2), full(be2)],
        out_specs=eb,
        out_shape=jax.ShapeDtypeStruct((E, H), jnp.float32),
    )(xr, xc, ea, w1a, w1b, w1c, b1, g1, be1, w2, b2, g2, be2)


def _tokens_tc(x, p):
    te = p["token_embed"]
    a = p["attn_tok"]
    args = [x, te,
            a["q"]["W"], a["q"]["b"][None, :], a["k"]["W"], a["k"]["b"][None, :],
            a["v"]["W"], a["v"]["b"][None, :], a["o"]["W"], a["o"]["b"][None, :],
            p["ln_tok1_g"][None, :], p["ln_tok1_b"][None, :],
            p["ff1"]["W"], p["ff1"]["b"][None, :],
            p["ff2"]["W"], p["ff2"]["b"][None, :],
            p["ln_tok2_g"][None, :], p["ln_tok2_b"][None, :]]

    def body(x_r, te_r, wq, bq, wk, bk, wv, bv, wo, bo, g1, be1,
             wf1, bf1, wf2, bf2, g2, be2, out_r):
        xx = x_r[...]
        tok = te_r[...]
        q = jnp.dot(tok, wq[...], preferred_element_type=jnp.float32) + bq[...]
        kk = jnp.dot(xx, wk[...], preferred_element_type=jnp.float32) + bk[...]
        vv = jnp.dot(xx, wv[...], preferred_element_type=jnp.float32) + bv[...]
        outs = []
        scale = 1.0 / math.sqrt(float(HD))
        for hh in range(NHEADS):
            sl = slice(hh * HD, (hh + 1) * HD)
            logits = lax.dot_general(q[:, sl], kk[:, sl],
                                     (((1,), (1,)), ((), ()))) * scale
            m = jnp.max(logits, axis=-1, keepdims=True)
            ex = jnp.exp(logits - m)
            pr = ex / jnp.sum(ex, axis=-1, keepdims=True)
            outs.append(lax.dot_general(pr, vv[:, sl], (((1,), (0,)), ((), ()))))
        o = jnp.concatenate(outs, axis=1)
        tok = tok + jnp.dot(o, wo[...], preferred_element_type=jnp.float32) + bo[...]
        tok = _ln(tok, g1[...], be1[...])
        f = _gelu(jnp.dot(tok, wf1[...], preferred_element_type=jnp.float32) + bf1[...])
        tok = tok + jnp.dot(f, wf2[...], preferred_element_type=jnp.float32) + bf2[...]
        tok = _ln(tok, g2[...], be2[...])
        out_r[...] = tok

    full = lambda arr: pl.BlockSpec(arr.shape, lambda: tuple(0 for _ in arr.shape))
    return pl.pallas_call(
        body,
        in_specs=[full(a_) for a_ in args],
        out_shape=jax.ShapeDtypeStruct((2, H), jnp.float32),
    )(*args)


N_B = 2000


def _node_tc(x, parts, cnt_parts, tokens, p_attn, p_mlp):
    W1 = p_mlp["lin1"]["W"]
    w1a, w1b, w1c = W1[:H], W1[H:2 * H], W1[2 * H:]
    args = [x, parts[0], parts[1], cnt_parts[0], cnt_parts[1], tokens,
            p_attn["q"]["W"], p_attn["q"]["b"][None, :],
            p_attn["k"]["W"], p_attn["k"]["b"][None, :],
            p_attn["v"]["W"], p_attn["v"]["b"][None, :],
            p_attn["o"]["W"], p_attn["o"]["b"][None, :],
            w1a, w1b, w1c, p_mlp["lin1"]["b"][None, :],
            p_mlp["ln1_g"][None, :], p_mlp["ln1_b"][None, :],
            p_mlp["lin2"]["W"], p_mlp["lin2"]["b"][None, :],
            p_mlp["ln2_g"][None, :], p_mlp["ln2_b"][None, :]]

    def body(x_r, p0_r, p1_r, c0_r, c1_r, tok_r,
             wq, bq, wk, bk, wv, bv, wo, bo,
             w1a_r, w1b_r, w1c_r, b1_r, g1, be1, w2, b2, g2, be2, out_r):
        xx = x_r[...]
        cnt = c0_r[...][:, 0:1] + c1_r[...][:, 0:1]
        agg = (p0_r[...] + p1_r[...]) / jnp.maximum(cnt, 1.0)
        tok = tok_r[...]
        q = jnp.dot(xx, wq[...], preferred_element_type=jnp.float32) + bq[...]
        tk = jnp.dot(tok, wk[...], preferred_element_type=jnp.float32) + bk[...]
        tv = jnp.dot(tok, wv[...], preferred_element_type=jnp.float32) + bv[...]
        scale = 1.0 / math.sqrt(float(HD))
        outs = []
        for hh in range(NHEADS):
            sl = slice(hh * HD, (hh + 1) * HD)
            logits = lax.dot_general(q[:, sl], tk[:, sl],
                                     (((1,), (1,)), ((), ()))) * scale
            m = jnp.max(logits, axis=-1, keepdims=True)
            ex = jnp.exp(logits - m)
            pr = ex / jnp.sum(ex, axis=-1, keepdims=True)
            outs.append(lax.dot_general(pr, tv[:, sl], (((1,), (0,)), ((), ()))))
        ctx = jnp.dot(jnp.concatenate(outs, axis=1), wo[...],
                      preferred_element_type=jnp.float32) + bo[...]
        h = jnp.dot(xx, w1a_r[...], preferred_element_type=jnp.float32)
        h = h + jnp.dot(agg, w1b_r[...], preferred_element_type=jnp.float32)
        h = h + jnp.dot(ctx, w1c_r[...], preferred_element_type=jnp.float32)
        h = h + b1_r[...]
        h = _gelu(_ln(h, g1[...], be1[...]))
        h2 = jnp.dot(h, w2[...], preferred_element_type=jnp.float32) + b2[...]
        h2 = _ln(h2, g2[...], be2[...])
        out_r[...] = xx + h2

    nb = pl.BlockSpec((N_B, H), lambda i: (i, 0))
    cb = pl.BlockSpec((N_B, H), lambda i: (i, 0))
    full = lambda arr: pl.BlockSpec(arr.shape, lambda i: tuple(0 for _ in arr.shape))
    specs = [nb, nb, nb, cb, cb] + [full(a_) for a_ in args[5:]]
    return pl.pallas_call(
        body,
        grid=(N // N_B,),
        in_specs=specs,
        out_specs=nb,
        out_shape=jax.ShapeDtypeStruct((N, H), jnp.float32),
    )(*args)


def kernel(x, edge_index, edge_attr, params):
    row = edge_index[0]
    col = edge_index[1]
    zeros_nh = jnp.zeros((NP, H), jnp.float32)
    ones_ch = jnp.ones((CHUNK, H), jnp.float32)

    cnt_parts = _count_sc(col, ones_ch, zeros_nh)
    for i in range(2):
        xp = jnp.pad(x, ((0, NP - N), (0, 0)))
        xr, xc = _gather_sc(xp, row, col)
        edge_attr = _edge_tc(xr, xc, edge_attr, params["edge"][i])
        parts = _scatter_sc(edge_attr, col, zeros_nh)
        tokens = _tokens_tc(x, params["gtt"])
        x = _node_tc(x, parts, cnt_parts, tokens,
                     params["gtt"]["attn_node"], params["node"][i])
    return x, edge_attr
